# baseline (speedup 1.0000x reference)
import jax
import jax.numpy as jnp
from jax import lax
from jax.experimental import pallas as pl
from jax.experimental.pallas import tpu as pltpu
from jax.experimental.pallas import tpu_sc as plsc

N = 10000
E = 320000
D = 128
D2 = D // 2
EPS = 1e-5
NEG_SLOPE = 0.01

NC = 2
NS = 16
NW = NC * NS
E_PER_T = E // NS
CHUNK = 100
NCH = E_PER_T // CHUNK
N_PAD = 10240
STRIPE = N_PAD // NS
DEGW = 16
ZROWS = 80
NBLK = STRIPE // ZROWS


def _node_mm_body(x_ref, wd_ref, w2_ref, b_ref, a_ref, bt_ref):
    x = x_ref[...]
    a_ref[...] = jnp.dot(x, wd_ref[...], preferred_element_type=jnp.float32) + b_ref[...]
    bb = jnp.dot(x, w2_ref[...], preferred_element_type=jnp.float32)
    bt_ref[0] = bb[:, :D2]
    bt_ref[1] = bb[:, D2:]


def _node_mm(x, wd_t, w2_t, b2d):
    return pl.pallas_call(
        _node_mm_body,
        out_shape=(
            jax.ShapeDtypeStruct((N, D), jnp.float32),
            jax.ShapeDtypeStruct((NC, N, D2), jnp.float32),
        ),
    )(x, wd_t, w2_t, b2d)


NBUF = 5
PREF = 2


def _edge_scatter_body(src_hbm, dst_hbm, bt_hbm,
                       s_out, d_out,
                       sidx, didx, rows0, rows1, rows2, rows3, rows4,
                       ones_v, zb_d,
                       gsem0, gsem1, gsem2, gsem3, gsem4,
                       ssem0, ssem1, ssem2, ssem3, ssem4, osem,
                       acc_s, acc_d):
    c = lax.axis_index("c")
    s = lax.axis_index("s")
    w = c * NS + s

    pltpu.sync_copy(src_hbm.at[s], sidx)
    pltpu.sync_copy(dst_hbm.at[s], didx)

    zero16 = jnp.zeros((16,), jnp.float32)
    one16 = jnp.ones((16,), jnp.float32)

    def fill_zs(i, carry):
        def inner(j, cc):
            rows0[i, pl.ds(pl.multiple_of(j * 16, 16), 16)] = zero16
            return cc
        return lax.fori_loop(0, D2 // 16, inner, carry)

    lax.fori_loop(0, ZROWS, fill_zs, 0)

    def fill_zd(i, carry):
        zb_d[i, :] = zero16
        return carry

    lax.fori_loop(0, ZROWS, fill_zd, 0)

    def fill_on(i, carry):
        ones_v[i, :] = one16
        return carry

    lax.fori_loop(0, CHUNK, fill_on, 0)

    for p in range(NBLK):
        pltpu.sync_copy(rows0.at[pl.ds(0, ZROWS)], acc_s.at[pl.ds(s * STRIPE + p * ZROWS, ZROWS)])
        pltpu.sync_copy(zb_d, acc_d.at[pl.ds(s * STRIPE + p * ZROWS, ZROWS)])

    plsc.subcore_barrier()

    rows_b = (rows0, rows1, rows2, rows3, rows4)
    gsems = (gsem0, gsem1, gsem2, gsem3, gsem4)
    ssems = (ssem0, ssem1, ssem2, ssem3, ssem4)
    LAG = NBUF - PREF

    def make_loop(half, deg_lo):
        def visit(k, b):
            pltpu.make_async_copy(bt_hbm.at[half].at[sidx.at[k]],
                                  rows_b[b], gsems[b]).wait()
            pltpu.async_copy(rows_b[b], acc_s.at[didx.at[k]], ssems[b],
                             add=True)

            @pl.when((k >= deg_lo) & (k < deg_lo + NCH // 2))
            def _deg():
                pltpu.async_copy(ones_v, acc_d.at[didx.at[k]], osem, add=True)

            bn = (b + PREF) % NBUF

            @pl.when(k >= LAG)
            def _free():
                pltpu.make_async_copy(rows_b[bn], acc_s.at[didx.at[0]],
                                      ssems[bn]).wait()

            @pl.when(k + PREF < NCH)
            def _prefetch():
                pltpu.async_copy(bt_hbm.at[half].at[sidx.at[k + PREF]],
                                 rows_b[bn], gsems[bn])

        def group(g, carry):
            for b in range(NBUF):
                visit(NBUF * g + b, b)
            return carry

        def run():
            for b in range(PREF):
                pltpu.async_copy(bt_hbm.at[half].at[sidx.at[b]],
                                 rows_b[b], gsems[b])
            lax.fori_loop(0, NCH // NBUF, group, 0)

        return run

    pl.when(c == 0)(make_loop(0, 0))
    pl.when(c == 1)(make_loop(1, NCH // 2))

    for k in range(NCH - LAG, NCH):
        b = k % NBUF
        pltpu.make_async_copy(rows_b[b], acc_s.at[didx.at[0]],
                              ssems[b]).wait()

    def drain_ones(i, carry):
        pltpu.make_async_copy(ones_v, acc_d.at[didx.at[0]], osem).wait()
        return carry

    lax.fori_loop(0, NCH // 2, drain_ones, 0)

    plsc.subcore_barrier()

    for p in range(NBLK):
        pltpu.sync_copy(acc_s.at[pl.ds(s * STRIPE + p * ZROWS, ZROWS)], rows0.at[pl.ds(0, ZROWS)])
        pltpu.sync_copy(rows0.at[pl.ds(0, ZROWS)], s_out.at[w, p])
        pltpu.sync_copy(acc_d.at[pl.ds(s * STRIPE + p * ZROWS, ZROWS)], zb_d)
        pltpu.sync_copy(zb_d, d_out.at[w, p])


def _edge_scatter(src3d, dst3d, bt):
    mesh = plsc.VectorSubcoreMesh(core_axis_name="c", subcore_axis_name="s")
    k = pl.kernel(
        _edge_scatter_body,
        out_type=(
            jax.ShapeDtypeStruct((NW, NBLK, ZROWS, D2), jnp.float32),
            jax.ShapeDtypeStruct((NW, NBLK, ZROWS, DEGW), jnp.float32),
        ),
        mesh=mesh,
        compiler_params=pltpu.CompilerParams(use_tc_tiling_on_sc=False),
        scratch_types=[
            pltpu.VMEM((NCH, CHUNK), jnp.int32),
            pltpu.VMEM((NCH, CHUNK), jnp.int32),
            pltpu.VMEM((CHUNK, D2), jnp.float32),
            pltpu.VMEM((CHUNK, D2), jnp.float32),
            pltpu.VMEM((CHUNK, D2), jnp.float32),
            pltpu.VMEM((CHUNK, D2), jnp.float32),
            pltpu.VMEM((CHUNK, D2), jnp.float32),
            pltpu.VMEM((CHUNK, DEGW), jnp.float32),
            pltpu.VMEM((ZROWS, DEGW), jnp.float32),
            pltpu.SemaphoreType.DMA,
            pltpu.SemaphoreType.DMA,
            pltpu.SemaphoreType.DMA,
            pltpu.SemaphoreType.DMA,
            pltpu.SemaphoreType.DMA,
            pltpu.SemaphoreType.DMA,
            pltpu.SemaphoreType.DMA,
            pltpu.SemaphoreType.DMA,
            pltpu.SemaphoreType.DMA,
            pltpu.SemaphoreType.DMA,
            pltpu.SemaphoreType.DMA,
            pltpu.VMEM_SHARED((N_PAD, D2), jnp.float32),
            pltpu.VMEM_SHARED((N_PAD, DEGW), jnp.float32),
        ],
    )
    return k(src3d, dst3d, bt)


def _finalize_body(a_ref, s0_ref, s1_ref, d0_ref, d1_ref, g_ref, be_ref, o_ref):
    s = jnp.concatenate([s0_ref[...], s1_ref[...]], axis=-1)
    deg = d0_ref[:, 0:1] + d1_ref[:, 0:1]
    h = deg * a_ref[...] + s
    mean = jnp.mean(h, axis=0, keepdims=True)
    var = jnp.mean((h - mean) ** 2, axis=0, keepdims=True)
    hn = (h - mean) * lax.rsqrt(var + EPS) * g_ref[...] + be_ref[...]
    o_ref[...] = jnp.where(hn >= 0, hn, NEG_SLOPE * hn)


def _finalize(a, s0, s1, d0, d1, gamma2d, beta2d):
    return pl.pallas_call(
        _finalize_body,
        out_shape=jax.ShapeDtypeStruct((N, D), jnp.float32),
    )(a, s0, s1, d0, d1, gamma2d, beta2d)


def kernel(node_features, edge_index, W, b, bn_weight, bn_bias):
    x = node_features.astype(jnp.float32)
    w1t = W[:, :D].T
    w2t = W[:, D:].T
    wd_t = w1t - w2t

    a, bt = _node_mm(x, wd_t, w2t, jnp.broadcast_to(b[None, :], (1, D)))

    src = edge_index[0].astype(jnp.int32).reshape(NS, NCH, CHUNK)
    dst = edge_index[1].astype(jnp.int32).reshape(NS, NCH, CHUNK)

    s_part, d_part = _edge_scatter(src, dst, bt)

    s0 = s_part[:NS].reshape(N_PAD, D2)[:N]
    s1 = s_part[NS:].reshape(N_PAD, D2)[:N]
    d0 = d_part[:NS].reshape(N_PAD, DEGW)[:N]
    d1 = d_part[NS:].reshape(N_PAD, DEGW)[:N]

    return _finalize(a, s0, s1, d0, d1,
                     jnp.broadcast_to(bn_weight[None, :], (1, D)),
                     jnp.broadcast_to(bn_bias[None, :], (1, D)))


# speedup vs baseline: 1.1426x; 1.1426x over previous
import jax
import jax.numpy as jnp
from jax import lax
from jax.experimental import pallas as pl
from jax.experimental.pallas import tpu as pltpu
from jax.experimental.pallas import tpu_sc as plsc

N = 10000
E = 320000
D = 128
D2 = D // 2
EPS = 1e-5
NEG_SLOPE = 0.01

NC = 2
NS = 16
NW = NC * NS
E_PER_T = E // NS
CHUNK = 80
NCH = E_PER_T // CHUNK
N_PAD = 10240
STRIPE = N_PAD // NS
DEGW = 16
ZROWS = 80
NBLK = STRIPE // ZROWS


def _node_mm_body(x_ref, wd_ref, w2_ref, b_ref, a_ref, bt_ref):
    x = x_ref[...]
    a_ref[...] = jnp.dot(x, wd_ref[...], preferred_element_type=jnp.float32) + b_ref[...]
    bb = jnp.dot(x, w2_ref[...], preferred_element_type=jnp.float32)
    bt_ref[0] = bb[:, :D2]
    bt_ref[1] = bb[:, D2:]


def _node_mm(x, wd_t, w2_t, b2d):
    return pl.pallas_call(
        _node_mm_body,
        out_shape=(
            jax.ShapeDtypeStruct((N, D), jnp.float32),
            jax.ShapeDtypeStruct((NC, N, D2), jnp.float32),
        ),
    )(x, wd_t, w2_t, b2d)


NBUF = 5
PREF = 2


def _edge_scatter_body(edge_hbm, bt_hbm,
                       s_out, d_out,
                       sidx, didx, rows0, rows1, rows2, rows3, rows4,
                       ones_v, zb_d,
                       gsem0, gsem1, gsem2, gsem3, gsem4,
                       ssem0, ssem1, ssem2, ssem3, ssem4, osem,
                       acc_s, acc_d):
    c = lax.axis_index("c")
    s = lax.axis_index("s")
    w = c * NS + s

    pltpu.sync_copy(edge_hbm.at[0, pl.ds(s * E_PER_T, E_PER_T)], sidx)
    pltpu.sync_copy(edge_hbm.at[1, pl.ds(s * E_PER_T, E_PER_T)], didx)

    zero16 = jnp.zeros((16,), jnp.float32)
    one16 = jnp.ones((16,), jnp.float32)

    def fill_zs(i, carry):
        def inner(j, cc):
            rows0[i, pl.ds(pl.multiple_of(j * 16, 16), 16)] = zero16
            return cc
        return lax.fori_loop(0, D2 // 16, inner, carry)

    lax.fori_loop(0, ZROWS, fill_zs, 0)

    def fill_zd(i, carry):
        zb_d[i, :] = zero16
        return carry

    lax.fori_loop(0, ZROWS, fill_zd, 0)

    def fill_on(i, carry):
        ones_v[i, :] = one16
        return carry

    lax.fori_loop(0, CHUNK, fill_on, 0)

    for p in range(NBLK):
        pltpu.sync_copy(rows0.at[pl.ds(0, ZROWS)], acc_s.at[pl.ds(s * STRIPE + p * ZROWS, ZROWS)])
        pltpu.sync_copy(zb_d, acc_d.at[pl.ds(s * STRIPE + p * ZROWS, ZROWS)])

    plsc.subcore_barrier()

    rows_b = (rows0, rows1, rows2, rows3, rows4)
    gsems = (gsem0, gsem1, gsem2, gsem3, gsem4)
    ssems = (ssem0, ssem1, ssem2, ssem3, ssem4)
    LAG = NBUF - PREF

    def make_loop(half, deg_lo):
        def visit(k, b):
            pltpu.make_async_copy(bt_hbm.at[half].at[sidx.at[pl.ds(k * CHUNK, CHUNK)]],
                                  rows_b[b], gsems[b]).wait()
            pltpu.async_copy(rows_b[b], acc_s.at[didx.at[pl.ds(k * CHUNK, CHUNK)]],
                             ssems[b], add=True)

            @pl.when((k >= deg_lo) & (k < deg_lo + NCH // 2))
            def _deg():
                pltpu.async_copy(ones_v, acc_d.at[didx.at[pl.ds(k * CHUNK, CHUNK)]],
                                     osem, add=True)

            bn = (b + PREF) % NBUF

            @pl.when(k >= LAG)
            def _free():
                pltpu.make_async_copy(rows_b[bn], acc_s.at[didx.at[pl.ds(0, CHUNK)]],
                                      ssems[bn]).wait()

            @pl.when(k + PREF < NCH)
            def _prefetch():
                pltpu.async_copy(bt_hbm.at[half].at[sidx.at[pl.ds((k + PREF) * CHUNK, CHUNK)]],
                                 rows_b[bn], gsems[bn])

        def group(g, carry):
            for b in range(NBUF):
                visit(NBUF * g + b, b)
            return carry

        def run():
            for b in range(PREF):
                pltpu.async_copy(bt_hbm.at[half].at[sidx.at[pl.ds(b * CHUNK, CHUNK)]],
                                 rows_b[b], gsems[b])
            lax.fori_loop(0, NCH // NBUF, group, 0)

        return run

    pl.when(c == 0)(make_loop(0, 0))
    pl.when(c == 1)(make_loop(1, NCH // 2))

    for k in range(NCH - LAG, NCH):
        b = k % NBUF
        pltpu.make_async_copy(rows_b[b], acc_s.at[didx.at[pl.ds(0, CHUNK)]],
                              ssems[b]).wait()

    def drain_ones(i, carry):
        pltpu.make_async_copy(ones_v, acc_d.at[didx.at[pl.ds(0, CHUNK)]], osem).wait()
        return carry

    lax.fori_loop(0, NCH // 2, drain_ones, 0)

    plsc.subcore_barrier()

    for p in range(NBLK):
        pltpu.sync_copy(acc_s.at[pl.ds(s * STRIPE + p * ZROWS, ZROWS)], rows0.at[pl.ds(0, ZROWS)])
        pltpu.sync_copy(rows0.at[pl.ds(0, ZROWS)], s_out.at[w, p])
        pltpu.sync_copy(acc_d.at[pl.ds(s * STRIPE + p * ZROWS, ZROWS)], zb_d)
        pltpu.sync_copy(zb_d, d_out.at[w, p])


def _edge_scatter(edges, bt):
    mesh = plsc.VectorSubcoreMesh(core_axis_name="c", subcore_axis_name="s")
    k = pl.kernel(
        _edge_scatter_body,
        out_type=(
            jax.ShapeDtypeStruct((NW, NBLK, ZROWS, D2), jnp.float32),
            jax.ShapeDtypeStruct((NW, NBLK, ZROWS, DEGW), jnp.float32),
        ),
        mesh=mesh,
        compiler_params=pltpu.CompilerParams(use_tc_tiling_on_sc=False),
        scratch_types=[
            pltpu.VMEM((E_PER_T,), jnp.int32),
            pltpu.VMEM((E_PER_T,), jnp.int32),
            pltpu.VMEM((CHUNK, D2), jnp.float32),
            pltpu.VMEM((CHUNK, D2), jnp.float32),
            pltpu.VMEM((CHUNK, D2), jnp.float32),
            pltpu.VMEM((CHUNK, D2), jnp.float32),
            pltpu.VMEM((CHUNK, D2), jnp.float32),
            pltpu.VMEM((CHUNK, DEGW), jnp.float32),
            pltpu.VMEM((ZROWS, DEGW), jnp.float32),
            pltpu.SemaphoreType.DMA,
            pltpu.SemaphoreType.DMA,
            pltpu.SemaphoreType.DMA,
            pltpu.SemaphoreType.DMA,
            pltpu.SemaphoreType.DMA,
            pltpu.SemaphoreType.DMA,
            pltpu.SemaphoreType.DMA,
            pltpu.SemaphoreType.DMA,
            pltpu.SemaphoreType.DMA,
            pltpu.SemaphoreType.DMA,
            pltpu.SemaphoreType.DMA,
            pltpu.VMEM_SHARED((N_PAD, D2), jnp.float32),
            pltpu.VMEM_SHARED((N_PAD, DEGW), jnp.float32),
        ],
    )
    return k(edges, bt)


def _finalize_body(a_ref, sp_ref, dp_ref, g_ref, be_ref, o_ref):
    sp = sp_ref[...]
    s0 = sp[:NS].reshape(N_PAD, D2)[:N]
    s1 = sp[NS:].reshape(N_PAD, D2)[:N]
    s = jnp.concatenate([s0, s1], axis=-1)
    dp = dp_ref[...]
    d0 = dp[:NS].reshape(N_PAD, DEGW)[:N]
    d1 = dp[NS:].reshape(N_PAD, DEGW)[:N]
    deg = d0[:, 0:1] + d1[:, 0:1]
    h = deg * a_ref[...] + s
    mean = jnp.mean(h, axis=0, keepdims=True)
    var = jnp.mean((h - mean) ** 2, axis=0, keepdims=True)
    hn = (h - mean) * lax.rsqrt(var + EPS) * g_ref[...] + be_ref[...]
    o_ref[...] = jnp.where(hn >= 0, hn, NEG_SLOPE * hn)


def _finalize(a, s_part, d_part, gamma2d, beta2d):
    return pl.pallas_call(
        _finalize_body,
        out_shape=jax.ShapeDtypeStruct((N, D), jnp.float32),
    )(a, s_part, d_part, gamma2d, beta2d)


def kernel(node_features, edge_index, W, b, bn_weight, bn_bias):
    x = node_features.astype(jnp.float32)
    w1t = W[:, :D].T
    w2t = W[:, D:].T
    wd_t = w1t - w2t

    a, bt = _node_mm(x, wd_t, w2t, jnp.broadcast_to(b[None, :], (1, D)))

    s_part, d_part = _edge_scatter(edge_index.astype(jnp.int32), bt)

    return _finalize(a, s_part, d_part,
                     jnp.broadcast_to(bn_weight[None, :], (1, D)),
                     jnp.broadcast_to(bn_bias[None, :], (1, D)))


# speedup vs baseline: 1.1680x; 1.0222x over previous
import jax
import jax.numpy as jnp
from jax import lax
from jax.experimental import pallas as pl
from jax.experimental.pallas import tpu as pltpu
from jax.experimental.pallas import tpu_sc as plsc

N = 10000
E = 320000
D = 128
D2 = D // 2
EPS = 1e-5
NEG_SLOPE = 0.01

NC = 2
NS = 16
NW = NC * NS
E_PER_T = E // NS
CHUNK = 80
NCH = E_PER_T // CHUNK
N_PAD = 10240
STRIPE = N_PAD // NS
DEGW = 16
ZROWS = 80
NBLK = STRIPE // ZROWS


NBUF = 5
PREF = 2


def _edge_scatter_body(src_hbm, dst_hbm, xt_hbm,
                       s_out, d_out,
                       sidx, didx, rows0, rows1, rows2, rows3, rows4,
                       ones_v, zb_d,
                       gsem0, gsem1, gsem2, gsem3, gsem4,
                       ssem0, ssem1, ssem2, ssem3, ssem4, osem,
                       acc_s, acc_d):
    c = lax.axis_index("c")
    s = lax.axis_index("s")
    w = c * NS + s

    pltpu.sync_copy(src_hbm.at[pl.ds(s * E_PER_T, E_PER_T)], sidx)
    pltpu.sync_copy(dst_hbm.at[pl.ds(s * E_PER_T, E_PER_T)], didx)

    def xform(i, carry):
        off = pl.multiple_of(i * 16, 16)
        sidx[pl.ds(off, 16)] = sidx[pl.ds(off, 16)] * 2 + c
        return carry

    lax.fori_loop(0, E_PER_T // 16, xform, 0)

    zero16 = jnp.zeros((16,), jnp.float32)
    one16 = jnp.ones((16,), jnp.float32)

    def fill_zs(i, carry):
        def inner(j, cc):
            rows0[i, pl.ds(pl.multiple_of(j * 16, 16), 16)] = zero16
            return cc
        return lax.fori_loop(0, D2 // 16, inner, carry)

    lax.fori_loop(0, ZROWS, fill_zs, 0)

    def fill_zd(i, carry):
        zb_d[i, :] = zero16
        return carry

    lax.fori_loop(0, ZROWS, fill_zd, 0)

    def fill_on(i, carry):
        ones_v[i, :] = one16
        return carry

    lax.fori_loop(0, CHUNK, fill_on, 0)

    for p in range(NBLK):
        pltpu.sync_copy(rows0.at[pl.ds(0, ZROWS)], acc_s.at[pl.ds(s * STRIPE + p * ZROWS, ZROWS)])
        pltpu.sync_copy(zb_d, acc_d.at[pl.ds(s * STRIPE + p * ZROWS, ZROWS)])

    plsc.subcore_barrier()

    rows_b = (rows0, rows1, rows2, rows3, rows4)
    gsems = (gsem0, gsem1, gsem2, gsem3, gsem4)
    ssems = (ssem0, ssem1, ssem2, ssem3, ssem4)
    LAG = NBUF - PREF

    def make_loop(deg_lo):
        def visit(k, b):
            pltpu.make_async_copy(xt_hbm.at[sidx.at[pl.ds(k * CHUNK, CHUNK)]],
                                  rows_b[b], gsems[b]).wait()
            pltpu.async_copy(rows_b[b], acc_s.at[didx.at[pl.ds(k * CHUNK, CHUNK)]],
                             ssems[b], add=True)

            @pl.when((k >= deg_lo) & (k < deg_lo + NCH // 2))
            def _deg():
                pltpu.async_copy(ones_v, acc_d.at[didx.at[pl.ds(k * CHUNK, CHUNK)]],
                                     osem, add=True)

            bn = (b + PREF) % NBUF

            @pl.when(k >= LAG)
            def _free():
                pltpu.make_async_copy(rows_b[bn], acc_s.at[didx.at[pl.ds(0, CHUNK)]],
                                      ssems[bn]).wait()

            @pl.when(k + PREF < NCH)
            def _prefetch():
                pltpu.async_copy(xt_hbm.at[sidx.at[pl.ds((k + PREF) * CHUNK, CHUNK)]],
                                 rows_b[bn], gsems[bn])

        def group(g, carry):
            for b in range(NBUF):
                visit(NBUF * g + b, b)
            return carry

        def run():
            for b in range(PREF):
                pltpu.async_copy(xt_hbm.at[sidx.at[pl.ds(b * CHUNK, CHUNK)]],
                                 rows_b[b], gsems[b])
            lax.fori_loop(0, NCH // NBUF, group, 0)

        return run

    pl.when(c == 0)(make_loop(0))
    pl.when(c == 1)(make_loop(NCH // 2))

    for k in range(NCH - LAG, NCH):
        b = k % NBUF
        pltpu.make_async_copy(rows_b[b], acc_s.at[didx.at[pl.ds(0, CHUNK)]],
                              ssems[b]).wait()

    def drain_ones(i, carry):
        pltpu.make_async_copy(ones_v, acc_d.at[didx.at[pl.ds(0, CHUNK)]], osem).wait()
        return carry

    lax.fori_loop(0, NCH // 2, drain_ones, 0)

    plsc.subcore_barrier()

    for p in range(NBLK):
        pltpu.sync_copy(acc_s.at[pl.ds(s * STRIPE + p * ZROWS, ZROWS)], rows0.at[pl.ds(0, ZROWS)])
        pltpu.sync_copy(rows0.at[pl.ds(0, ZROWS)], s_out.at[w, p])
        pltpu.sync_copy(acc_d.at[pl.ds(s * STRIPE + p * ZROWS, ZROWS)], zb_d)
        pltpu.sync_copy(zb_d, d_out.at[w, p])


def _edge_scatter(src, dst, xt):
    mesh = plsc.VectorSubcoreMesh(core_axis_name="c", subcore_axis_name="s")
    k = pl.kernel(
        _edge_scatter_body,
        out_type=(
            jax.ShapeDtypeStruct((NW, NBLK, ZROWS, D2), jnp.float32),
            jax.ShapeDtypeStruct((NW, NBLK, ZROWS, DEGW), jnp.float32),
        ),
        mesh=mesh,
        compiler_params=pltpu.CompilerParams(use_tc_tiling_on_sc=False),
        scratch_types=[
            pltpu.VMEM((E_PER_T,), jnp.int32),
            pltpu.VMEM((E_PER_T,), jnp.int32),
            pltpu.VMEM((CHUNK, D2), jnp.float32),
            pltpu.VMEM((CHUNK, D2), jnp.float32),
            pltpu.VMEM((CHUNK, D2), jnp.float32),
            pltpu.VMEM((CHUNK, D2), jnp.float32),
            pltpu.VMEM((CHUNK, D2), jnp.float32),
            pltpu.VMEM((CHUNK, DEGW), jnp.float32),
            pltpu.VMEM((ZROWS, DEGW), jnp.float32),
            pltpu.SemaphoreType.DMA,
            pltpu.SemaphoreType.DMA,
            pltpu.SemaphoreType.DMA,
            pltpu.SemaphoreType.DMA,
            pltpu.SemaphoreType.DMA,
            pltpu.SemaphoreType.DMA,
            pltpu.SemaphoreType.DMA,
            pltpu.SemaphoreType.DMA,
            pltpu.SemaphoreType.DMA,
            pltpu.SemaphoreType.DMA,
            pltpu.SemaphoreType.DMA,
            pltpu.VMEM_SHARED((N_PAD, D2), jnp.float32),
            pltpu.VMEM_SHARED((N_PAD, DEGW), jnp.float32),
        ],
    )
    return k(src, dst, xt)


def _finalize_body(x_ref, sp_ref, dp_ref, wd_ref, w2_ref, b_ref, g_ref, be_ref, o_ref):
    sp = sp_ref[...]
    s0 = sp[:NS].reshape(N_PAD, D2)[:N]
    s1 = sp[NS:].reshape(N_PAD, D2)[:N]
    xa = jnp.concatenate([s0, s1], axis=-1)
    dp = dp_ref[...]
    d0 = dp[:NS].reshape(N_PAD, DEGW)[:N]
    d1 = dp[NS:].reshape(N_PAD, DEGW)[:N]
    deg = d0[:, 0:1] + d1[:, 0:1]
    a = jnp.dot(x_ref[...], wd_ref[...], preferred_element_type=jnp.float32) + b_ref[...]
    s = jnp.dot(xa, w2_ref[...], preferred_element_type=jnp.float32)
    h = deg * a + s
    mean = jnp.mean(h, axis=0, keepdims=True)
    var = jnp.mean((h - mean) ** 2, axis=0, keepdims=True)
    hn = (h - mean) * lax.rsqrt(var + EPS) * g_ref[...] + be_ref[...]
    o_ref[...] = jnp.where(hn >= 0, hn, NEG_SLOPE * hn)


def _finalize(x, s_part, d_part, wd_t, w2_t, b2d, gamma2d, beta2d):
    return pl.pallas_call(
        _finalize_body,
        out_shape=jax.ShapeDtypeStruct((N, D), jnp.float32),
    )(x, s_part, d_part, wd_t, w2_t, b2d, gamma2d, beta2d)


def kernel(node_features, edge_index, W, b, bn_weight, bn_bias):
    x = node_features.astype(jnp.float32)
    w1t = W[:, :D].T
    w2t = W[:, D:].T
    wd_t = w1t - w2t

    xt = x.reshape(2 * N, D2)
    ei = edge_index.astype(jnp.int32)
    s_part, d_part = _edge_scatter(ei[0], ei[1], xt)

    return _finalize(x, s_part, d_part, wd_t, w2t,
                     jnp.broadcast_to(b[None, :], (1, D)),
                     jnp.broadcast_to(bn_weight[None, :], (1, D)),
                     jnp.broadcast_to(bn_bias[None, :], (1, D)))


# speedup vs baseline: 1.4100x; 1.2072x over previous
import jax
import jax.numpy as jnp
from jax import lax
from jax.experimental import pallas as pl
from jax.experimental.pallas import tpu as pltpu
from jax.experimental.pallas import tpu_sc as plsc

N = 10000
E = 320000
D = 128
D2 = D // 2
EPS = 1e-5
NEG_SLOPE = 0.01

NC = 2
NS = 16
NW = NC * NS
E_PER_T = E // NS
CHUNK = 80
NCH = E_PER_T // CHUNK
N_PAD = 10240
STRIPE = N_PAD // NS
DEGW = 16
ZROWS = 80
NBLK = STRIPE // ZROWS


NBUF = 5
PREF = 3


def _edge_scatter_body(src_hbm, dst_hbm, xt_hbm,
                       s_out, d_out,
                       sidx, didx, rows0, rows1, rows2, rows3, rows4,
                       ones_v, zb_d,
                       gsem0, gsem1, gsem2, gsem3, gsem4,
                       ssem0, ssem1, ssem2, ssem3, ssem4, osem,
                       acc_s, acc_d):
    c = lax.axis_index("c")
    s = lax.axis_index("s")
    w = c * NS + s

    pltpu.sync_copy(src_hbm.at[pl.ds(s * E_PER_T, E_PER_T)], sidx)
    pltpu.sync_copy(dst_hbm.at[pl.ds(s * E_PER_T, E_PER_T)], didx)


    def xform_chunk(k):
        for j in range(CHUNK // 16):
            off = pl.multiple_of(k * CHUNK + j * 16, 16)
            sidx[pl.ds(off, 16)] = sidx[pl.ds(off, 16)] * 2 + c

    zero16 = jnp.zeros((16,), jnp.float32)
    one16 = jnp.ones((16,), jnp.float32)

    def fill_zs(i, carry):
        def inner(j, cc):
            rows0[i, pl.ds(pl.multiple_of(j * 16, 16), 16)] = zero16
            return cc
        return lax.fori_loop(0, D2 // 16, inner, carry)

    lax.fori_loop(0, ZROWS, fill_zs, 0)

    def fill_zd(i, carry):
        zb_d[i, :] = zero16
        return carry

    lax.fori_loop(0, ZROWS, fill_zd, 0)

    def fill_on(i, carry):
        ones_v[i, :] = one16
        return carry

    lax.fori_loop(0, CHUNK, fill_on, 0)

    for p in range(NBLK):
        pltpu.sync_copy(rows0.at[pl.ds(0, ZROWS)], acc_s.at[pl.ds(s * STRIPE + p * ZROWS, ZROWS)])
        pltpu.sync_copy(zb_d, acc_d.at[pl.ds(s * STRIPE + p * ZROWS, ZROWS)])

    plsc.subcore_barrier()

    rows_b = (rows0, rows1, rows2, rows3, rows4)
    gsems = (gsem0, gsem1, gsem2, gsem3, gsem4)
    ssems = (ssem0, ssem1, ssem2, ssem3, ssem4)
    LAG = NBUF - PREF

    def make_loop(deg_lo):
        def visit(k, b):
            pltpu.make_async_copy(xt_hbm.at[sidx.at[pl.ds(k * CHUNK, CHUNK)]],
                                  rows_b[b], gsems[b]).wait()
            pltpu.async_copy(rows_b[b], acc_s.at[didx.at[pl.ds(k * CHUNK, CHUNK)]],
                             ssems[b], add=True)

            @pl.when((k >= deg_lo) & (k < deg_lo + NCH // 2))
            def _deg():
                pltpu.async_copy(ones_v, acc_d.at[didx.at[pl.ds(k * CHUNK, CHUNK)]],
                                     osem, add=True)

            bn = (b + PREF) % NBUF

            @pl.when(k >= LAG)
            def _free():
                pltpu.make_async_copy(rows_b[bn], acc_s.at[didx.at[pl.ds(0, CHUNK)]],
                                      ssems[bn]).wait()

            @pl.when(k + PREF < NCH)
            def _prefetch():
                xform_chunk(k + PREF)
                pltpu.async_copy(xt_hbm.at[sidx.at[pl.ds((k + PREF) * CHUNK, CHUNK)]],
                                 rows_b[bn], gsems[bn])

        def group(g, carry):
            for b in range(NBUF):
                visit(NBUF * g + b, b)
            return carry

        def run():
            for b in range(PREF):
                xform_chunk(b)
                pltpu.async_copy(xt_hbm.at[sidx.at[pl.ds(b * CHUNK, CHUNK)]],
                                 rows_b[b], gsems[b])
            lax.fori_loop(0, NCH // NBUF, group, 0)

        return run

    pl.when(c == 0)(make_loop(0))
    pl.when(c == 1)(make_loop(NCH // 2))

    for k in range(NCH - LAG, NCH):
        b = k % NBUF
        pltpu.make_async_copy(rows_b[b], acc_s.at[didx.at[pl.ds(0, CHUNK)]],
                              ssems[b]).wait()

    def drain_ones(i, carry):
        pltpu.make_async_copy(ones_v, acc_d.at[didx.at[pl.ds(0, CHUNK)]], osem).wait()
        return carry

    lax.fori_loop(0, NCH // 2, drain_ones, 0)

    plsc.subcore_barrier()

    for p in range(NBLK):
        pltpu.sync_copy(acc_s.at[pl.ds(s * STRIPE + p * ZROWS, ZROWS)], rows0.at[pl.ds(0, ZROWS)])
        pltpu.sync_copy(rows0.at[pl.ds(0, ZROWS)], s_out.at[w, p])
        pltpu.sync_copy(acc_d.at[pl.ds(s * STRIPE + p * ZROWS, ZROWS)], zb_d)
        pltpu.sync_copy(zb_d, d_out.at[w, p])


def _edge_scatter(src, dst, xt):
    mesh = plsc.VectorSubcoreMesh(core_axis_name="c", subcore_axis_name="s")
    k = pl.kernel(
        _edge_scatter_body,
        out_type=(
            jax.ShapeDtypeStruct((NW, NBLK, ZROWS, D2), jnp.float32),
            jax.ShapeDtypeStruct((NW, NBLK, ZROWS, DEGW), jnp.float32),
        ),
        mesh=mesh,
        compiler_params=pltpu.CompilerParams(use_tc_tiling_on_sc=False),
        scratch_types=[
            pltpu.VMEM((E_PER_T,), jnp.int32),
            pltpu.VMEM((E_PER_T,), jnp.int32),
            pltpu.VMEM((CHUNK, D2), jnp.float32),
            pltpu.VMEM((CHUNK, D2), jnp.float32),
            pltpu.VMEM((CHUNK, D2), jnp.float32),
            pltpu.VMEM((CHUNK, D2), jnp.float32),
            pltpu.VMEM((CHUNK, D2), jnp.float32),
            pltpu.VMEM((CHUNK, DEGW), jnp.float32),
            pltpu.VMEM((ZROWS, DEGW), jnp.float32),
            pltpu.SemaphoreType.DMA,
            pltpu.SemaphoreType.DMA,
            pltpu.SemaphoreType.DMA,
            pltpu.SemaphoreType.DMA,
            pltpu.SemaphoreType.DMA,
            pltpu.SemaphoreType.DMA,
            pltpu.SemaphoreType.DMA,
            pltpu.SemaphoreType.DMA,
            pltpu.SemaphoreType.DMA,
            pltpu.SemaphoreType.DMA,
            pltpu.SemaphoreType.DMA,
            pltpu.VMEM_SHARED((N_PAD, D2), jnp.float32),
            pltpu.VMEM_SHARED((N_PAD, DEGW), jnp.float32),
        ],
    )
    return k(src, dst, xt)


def _finalize_body(x_ref, sp_ref, dp_ref, wd_ref, w2_ref, b_ref, g_ref, be_ref, o_ref):
    sp = sp_ref[...]
    s0 = sp[:NS].reshape(N_PAD, D2)[:N]
    s1 = sp[NS:].reshape(N_PAD, D2)[:N]
    xa = jnp.concatenate([s0, s1], axis=-1)
    dp = dp_ref[...]
    d0 = dp[:NS].reshape(N_PAD, DEGW)[:N]
    d1 = dp[NS:].reshape(N_PAD, DEGW)[:N]
    deg = d0[:, 0:1] + d1[:, 0:1]
    a = jnp.dot(x_ref[...], wd_ref[...], preferred_element_type=jnp.float32) + b_ref[...]
    s = jnp.dot(xa, w2_ref[...], preferred_element_type=jnp.float32)
    h = deg * a + s
    mean = jnp.mean(h, axis=0, keepdims=True)
    var = jnp.mean((h - mean) ** 2, axis=0, keepdims=True)
    hn = (h - mean) * lax.rsqrt(var + EPS) * g_ref[...] + be_ref[...]
    o_ref[...] = jnp.where(hn >= 0, hn, NEG_SLOPE * hn)


def _finalize(x, s_part, d_part, wd_t, w2_t, b2d, gamma2d, beta2d):
    return pl.pallas_call(
        _finalize_body,
        out_shape=jax.ShapeDtypeStruct((N, D), jnp.float32),
    )(x, s_part, d_part, wd_t, w2_t, b2d, gamma2d, beta2d)


def kernel(node_features, edge_index, W, b, bn_weight, bn_bias):
    x = node_features.astype(jnp.float32)
    w1t = W[:, :D].T
    w2t = W[:, D:].T
    wd_t = w1t - w2t

    xt = x.reshape(2 * N, D2)
    ei = edge_index.astype(jnp.int32)
    s_part, d_part = _edge_scatter(ei[0], ei[1], xt)

    return _finalize(x, s_part, d_part, wd_t, w2t,
                     jnp.broadcast_to(b[None, :], (1, D)),
                     jnp.broadcast_to(bn_weight[None, :], (1, D)),
                     jnp.broadcast_to(bn_bias[None, :], (1, D)))


# speedup vs baseline: 1.4700x; 1.0425x over previous
import jax
import jax.numpy as jnp
from jax import lax
from jax.experimental import pallas as pl
from jax.experimental.pallas import tpu as pltpu
from jax.experimental.pallas import tpu_sc as plsc

N = 10000
E = 320000
D = 128
D2 = D // 2
EPS = 1e-5
NEG_SLOPE = 0.01

NC = 2
NS = 16
NW = NC * NS
E_PER_T = E // NS
CHUNK = 80
NCH = E_PER_T // CHUNK
N_PAD = 10240
STRIPE = N_PAD // NS
DEGW = 16
ZROWS = 80
NBLK = STRIPE // ZROWS


NBUF = 5
PREF = 4


def _edge_scatter_body(src_hbm, dst_hbm, xt_hbm,
                       s_out, d_out,
                       sidx, didx, rows0, rows1, rows2, rows3, rows4,
                       ones_v, zb_d,
                       gsem0, gsem1, gsem2, gsem3, gsem4,
                       ssem0, ssem1, ssem2, ssem3, ssem4, osem,
                       acc_s, acc_d):
    c = lax.axis_index("c")
    s = lax.axis_index("s")
    w = c * NS + s

    pltpu.sync_copy(src_hbm.at[pl.ds(s * E_PER_T, E_PER_T)], sidx)
    pltpu.sync_copy(dst_hbm.at[pl.ds(s * E_PER_T, E_PER_T)], didx)


    def xform_chunk(k):
        for j in range(CHUNK // 16):
            off = pl.multiple_of(k * CHUNK + j * 16, 16)
            sidx[pl.ds(off, 16)] = sidx[pl.ds(off, 16)] * 2 + c

    zero16 = jnp.zeros((16,), jnp.float32)
    one16 = jnp.ones((16,), jnp.float32)

    def fill_zs(i, carry):
        def inner(j, cc):
            rows0[i, pl.ds(pl.multiple_of(j * 16, 16), 16)] = zero16
            return cc
        return lax.fori_loop(0, D2 // 16, inner, carry)

    lax.fori_loop(0, ZROWS, fill_zs, 0)

    def fill_zd(i, carry):
        zb_d[i, :] = zero16
        return carry

    lax.fori_loop(0, ZROWS, fill_zd, 0)

    def fill_on(i, carry):
        ones_v[i, :] = one16
        return carry

    lax.fori_loop(0, CHUNK, fill_on, 0)

    for p in range(NBLK):
        pltpu.sync_copy(rows0.at[pl.ds(0, ZROWS)], acc_s.at[pl.ds(s * STRIPE + p * ZROWS, ZROWS)])
        pltpu.sync_copy(zb_d, acc_d.at[pl.ds(s * STRIPE + p * ZROWS, ZROWS)])

    plsc.subcore_barrier()

    rows_b = (rows0, rows1, rows2, rows3, rows4)
    gsems = (gsem0, gsem1, gsem2, gsem3, gsem4)
    ssems = (ssem0, ssem1, ssem2, ssem3, ssem4)
    LAG = NBUF - PREF

    def make_loop(deg_lo):
        def visit(k, b):
            pltpu.make_async_copy(xt_hbm.at[sidx.at[pl.ds(k * CHUNK, CHUNK)]],
                                  rows_b[b], gsems[b]).wait()
            pltpu.async_copy(rows_b[b], acc_s.at[didx.at[pl.ds(k * CHUNK, CHUNK)]],
                             ssems[b], add=True)

            @pl.when((k >= deg_lo) & (k < deg_lo + NCH // 2))
            def _deg():
                pltpu.async_copy(ones_v, acc_d.at[didx.at[pl.ds(k * CHUNK, CHUNK)]],
                                     osem, add=True)

            bn = (b + PREF) % NBUF

            @pl.when(k >= LAG)
            def _free():
                pltpu.make_async_copy(rows_b[bn], acc_s.at[didx.at[pl.ds(0, CHUNK)]],
                                      ssems[bn]).wait()

            @pl.when(k + PREF < NCH)
            def _prefetch():
                xform_chunk(k + PREF)
                pltpu.async_copy(xt_hbm.at[sidx.at[pl.ds((k + PREF) * CHUNK, CHUNK)]],
                                 rows_b[bn], gsems[bn])

        def group(g, carry):
            for b in range(NBUF):
                visit(NBUF * g + b, b)
            return carry

        def run():
            for b in range(PREF):
                xform_chunk(b)
                pltpu.async_copy(xt_hbm.at[sidx.at[pl.ds(b * CHUNK, CHUNK)]],
                                 rows_b[b], gsems[b])
            lax.fori_loop(0, NCH // NBUF, group, 0)

        return run

    pl.when(c == 0)(make_loop(0))
    pl.when(c == 1)(make_loop(NCH // 2))

    for k in range(NCH - LAG, NCH):
        b = k % NBUF
        pltpu.make_async_copy(rows_b[b], acc_s.at[didx.at[pl.ds(0, CHUNK)]],
                              ssems[b]).wait()

    def drain_ones(i, carry):
        pltpu.make_async_copy(ones_v, acc_d.at[didx.at[pl.ds(0, CHUNK)]], osem).wait()
        return carry

    lax.fori_loop(0, NCH // 2, drain_ones, 0)

    plsc.subcore_barrier()

    for p in range(NBLK):
        pltpu.sync_copy(acc_s.at[pl.ds(s * STRIPE + p * ZROWS, ZROWS)], rows0.at[pl.ds(0, ZROWS)])
        pltpu.sync_copy(rows0.at[pl.ds(0, ZROWS)], s_out.at[w, p])
        pltpu.sync_copy(acc_d.at[pl.ds(s * STRIPE + p * ZROWS, ZROWS)], zb_d)
        pltpu.sync_copy(zb_d, d_out.at[w, p])


def _edge_scatter(src, dst, xt):
    mesh = plsc.VectorSubcoreMesh(core_axis_name="c", subcore_axis_name="s")
    k = pl.kernel(
        _edge_scatter_body,
        out_type=(
            jax.ShapeDtypeStruct((NW, NBLK, ZROWS, D2), jnp.float32),
            jax.ShapeDtypeStruct((NW, NBLK, ZROWS, DEGW), jnp.float32),
        ),
        mesh=mesh,
        compiler_params=pltpu.CompilerParams(use_tc_tiling_on_sc=False),
        scratch_types=[
            pltpu.VMEM((E_PER_T,), jnp.int32),
            pltpu.VMEM((E_PER_T,), jnp.int32),
            pltpu.VMEM((CHUNK, D2), jnp.float32),
            pltpu.VMEM((CHUNK, D2), jnp.float32),
            pltpu.VMEM((CHUNK, D2), jnp.float32),
            pltpu.VMEM((CHUNK, D2), jnp.float32),
            pltpu.VMEM((CHUNK, D2), jnp.float32),
            pltpu.VMEM((CHUNK, DEGW), jnp.float32),
            pltpu.VMEM((ZROWS, DEGW), jnp.float32),
            pltpu.SemaphoreType.DMA,
            pltpu.SemaphoreType.DMA,
            pltpu.SemaphoreType.DMA,
            pltpu.SemaphoreType.DMA,
            pltpu.SemaphoreType.DMA,
            pltpu.SemaphoreType.DMA,
            pltpu.SemaphoreType.DMA,
            pltpu.SemaphoreType.DMA,
            pltpu.SemaphoreType.DMA,
            pltpu.SemaphoreType.DMA,
            pltpu.SemaphoreType.DMA,
            pltpu.VMEM_SHARED((N_PAD, D2), jnp.float32),
            pltpu.VMEM_SHARED((N_PAD, DEGW), jnp.float32),
        ],
    )
    return k(src, dst, xt)


def _finalize_body(x_ref, sp_ref, dp_ref, wd_ref, w2_ref, b_ref, g_ref, be_ref, o_ref):
    sp = sp_ref[...]
    s0 = sp[:NS].reshape(N_PAD, D2)[:N]
    s1 = sp[NS:].reshape(N_PAD, D2)[:N]
    xa = jnp.concatenate([s0, s1], axis=-1)
    dp = dp_ref[...]
    d0 = dp[:NS].reshape(N_PAD, DEGW)[:N]
    d1 = dp[NS:].reshape(N_PAD, DEGW)[:N]
    deg = d0[:, 0:1] + d1[:, 0:1]
    a = jnp.dot(x_ref[...], wd_ref[...], preferred_element_type=jnp.float32) + b_ref[...]
    s = jnp.dot(xa, w2_ref[...], preferred_element_type=jnp.float32)
    h = deg * a + s
    mean = jnp.mean(h, axis=0, keepdims=True)
    var = jnp.mean((h - mean) ** 2, axis=0, keepdims=True)
    hn = (h - mean) * lax.rsqrt(var + EPS) * g_ref[...] + be_ref[...]
    o_ref[...] = jnp.where(hn >= 0, hn, NEG_SLOPE * hn)


def _finalize(x, s_part, d_part, wd_t, w2_t, b2d, gamma2d, beta2d):
    return pl.pallas_call(
        _finalize_body,
        out_shape=jax.ShapeDtypeStruct((N, D), jnp.float32),
    )(x, s_part, d_part, wd_t, w2_t, b2d, gamma2d, beta2d)


def kernel(node_features, edge_index, W, b, bn_weight, bn_bias):
    x = node_features.astype(jnp.float32)
    w1t = W[:, :D].T
    w2t = W[:, D:].T
    wd_t = w1t - w2t

    xt = x.reshape(2 * N, D2)
    ei = edge_index.astype(jnp.int32)
    s_part, d_part = _edge_scatter(ei[0], ei[1], xt)

    return _finalize(x, s_part, d_part, wd_t, w2t,
                     jnp.broadcast_to(b[None, :], (1, D)),
                     jnp.broadcast_to(bn_weight[None, :], (1, D)),
                     jnp.broadcast_to(bn_bias[None, :], (1, D)))
